# baseline (speedup 1.0000x reference)
import jax
import jax.numpy as jnp
from jax import lax
from jax.experimental import pallas as pl
from jax.experimental.pallas import tpu as pltpu
from jax.experimental.pallas import tpu_sc as plsc

MAXLEN = 200
EMBED = 64
BATCH = 4096
NC = 2
NS = 16
NW = NC * NS
SEQ_PER_W = BATCH // NW
SPC = 4
NCHUNK = SEQ_PER_W // SPC
PARTS = ((0, 104), (104, 96))


def _body(x, tok, pos, out3, xb0, xb1, rows0, rows1, pos_v,
          gs0, gs1, ss0, ss1):
    wid = lax.axis_index("s") * NC + lax.axis_index("c")
    seq_base = wid * SEQ_PER_W
    pltpu.sync_copy(pos, pos_v)

    bufs = ((xb0, rows0, gs0, ss0), (xb1, rows1, gs1, ss1))

    def fire(g, b, wait_store):
        xblk, rows_v, gsem, ssem = bufs[b]
        gg = lax.min(g, NCHUNK - 1)
        seq0 = seq_base + gg * SPC
        if wait_store:
            pltpu.make_async_copy(rows_v, out3.at[pl.ds(seq0, SPC)],
                                  ssem).wait()
        pltpu.sync_copy(x.at[pl.ds(seq0, SPC)], xblk)
        for s in range(SPC):
            for o, n in PARTS:
                pltpu.async_copy(
                    tok.at[xblk.at[s, pl.ds(o, n)]],
                    rows_v.at[s, pl.ds(o, n)], gsem)

    def proc(g, b):
        xblk, rows_v, gsem, ssem = bufs[b]
        seq0 = seq_base + g * SPC
        for s in range(SPC):
            for o, n in PARTS:
                pltpu.make_async_copy(
                    tok.at[xblk.at[s, pl.ds(o, n)]],
                    rows_v.at[s, pl.ds(o, n)], gsem).wait()

        def addl(l, carry):
            for d in range(EMBED // 16):
                sl = pl.ds(d * 16, 16)
                pv = pos_v[l, sl]
                for s in range(SPC):
                    rows_v[s, l, sl] = rows_v[s, l, sl] + pv
            return carry

        lax.fori_loop(0, MAXLEN, addl, 0)
        pltpu.async_copy(rows_v, out3.at[pl.ds(seq0, SPC)], ssem)

    fire(0, 0, False)
    fire(1, 1, False)

    def pair(h, carry):
        g = 2 * h
        proc(g, 0)
        fire(g + 2, 0, True)
        proc(g + 1, 1)
        fire(g + 3, 1, True)
        return carry

    lax.fori_loop(0, NCHUNK // 2, pair, 0)

    for b in (0, 1):
        xblk, rows_v, gsem, _ = bufs[b]
        for s in range(SPC):
            for o, n in PARTS:
                pltpu.make_async_copy(
                    tok.at[xblk.at[s, pl.ds(o, n)]],
                    rows_v.at[s, pl.ds(o, n)], gsem).wait()


@jax.jit
def _run(x, tok, pos):
    mesh = plsc.VectorSubcoreMesh(core_axis_name="c", subcore_axis_name="s")
    f = pl.kernel(
        _body,
        out_type=jax.ShapeDtypeStruct((BATCH, MAXLEN, EMBED), jnp.float32),
        mesh=mesh,
        scratch_types=[
            pltpu.VMEM((SPC, MAXLEN), jnp.int32),
            pltpu.VMEM((SPC, MAXLEN), jnp.int32),
            pltpu.VMEM((SPC, MAXLEN, EMBED), jnp.float32),
            pltpu.VMEM((SPC, MAXLEN, EMBED), jnp.float32),
            pltpu.VMEM((MAXLEN, EMBED), jnp.float32),
            pltpu.SemaphoreType.DMA,
            pltpu.SemaphoreType.DMA,
            pltpu.SemaphoreType.DMA,
            pltpu.SemaphoreType.DMA,
        ],
        compiler_params=pltpu.CompilerParams(use_tc_tiling_on_sc=False),
    )
    return f(x, tok, pos)


def kernel(x, token_table, pos_table):
    return _run(x.astype(jnp.int32), token_table, pos_table)
